# baseline (speedup 1.0000x reference)
import functools

import jax
import jax.numpy as jnp
from jax import lax
from jax.experimental import pallas as pl
from jax.experimental.pallas import tpu as pltpu
from jax.experimental.pallas import tpu_sc as plsc

N = 10000
D = 128
E = 320000
OUT = 2

NC = 2
NS = 16
NW = NC * NS

K = 128
BCH = 20
TOT_CH = 160
CH0 = 80
CH1 = TOT_CH - CH0
E_PAD = NS * TOT_CH * K
NPAD = 10112
ROWS_PER_SUB = NPAD // NS

_sc_mesh = plsc.VectorSubcoreMesh(core_axis_name="c", subcore_axis_name="s")


def _agg_body(table, idx_i, zero_hbm, out, acc, ibuf0, ibuf1,
              rows0, rows1, isem0, isem1, sem0, sem1):
    c = lax.axis_index("c")
    s = lax.axis_index("s")
    pltpu.sync_copy(zero_hbm, acc.at[pl.ds(s * ROWS_PER_SUB, ROWS_PER_SUB)])
    plsc.subcore_barrier()

    ibufs = (ibuf0, ibuf1)
    isems = (isem0, isem1)

    def run(ch_lo, ch_hi):
        nblk = (ch_hi - ch_lo) // BCH
        pltpu.async_copy(idx_i.at[s, pl.ds(ch_lo, BCH)], ibuf0, isem0)
        for b in range(nblk):
            ib = ibufs[b % 2]
            pltpu.make_async_copy(idx_i.at[s, pl.ds(ch_lo + b * BCH, BCH)],
                                  ib, isems[b % 2]).wait()
            if b + 1 < nblk:
                pltpu.async_copy(
                    idx_i.at[s, pl.ds(ch_lo + (b + 1) * BCH, BCH)],
                    ibufs[(b + 1) % 2], isems[(b + 1) % 2])

            pltpu.async_copy(table.at[ib.at[0, 0]], rows0, sem0)

            def pair(j, carry):
                i0 = 2 * j
                pltpu.async_copy(table.at[ib.at[i0 + 1, 0]], rows1, sem1)
                pltpu.make_async_copy(table.at[ib.at[i0, 0]], rows0, sem0).wait()
                pltpu.sync_copy(rows0, acc.at[ib.at[i0, 1]], add=True)

                @pl.when(j < BCH // 2 - 1)
                def _():
                    pltpu.async_copy(table.at[ib.at[i0 + 2, 0]], rows0, sem0)

                pltpu.make_async_copy(table.at[ib.at[i0 + 1, 0]], rows1,
                                      sem1).wait()
                pltpu.sync_copy(rows1, acc.at[ib.at[i0 + 1, 1]], add=True)
                return carry

            lax.fori_loop(0, BCH // 2, pair, 0)

    if CH0 > 0:
        @pl.when(c == 0)
        def _():
            run(0, CH0)
    if CH1 > 0:
        @pl.when(c == 1)
        def _():
            run(CH0, TOT_CH)

    plsc.subcore_barrier()
    pltpu.sync_copy(acc.at[pl.ds(s * ROWS_PER_SUB, ROWS_PER_SUB)],
                    out.at[c, pl.ds(s * ROWS_PER_SUB, ROWS_PER_SUB)])


_agg_call = functools.partial(
    pl.kernel,
    _agg_body,
    out_type=jax.ShapeDtypeStruct((NC, NPAD, D), jnp.float32),
    mesh=_sc_mesh,
    scratch_types=[
        pltpu.VMEM_SHARED((NPAD, D), jnp.float32),
        pltpu.VMEM((BCH, 2, K), jnp.int32),
        pltpu.VMEM((BCH, 2, K), jnp.int32),
        pltpu.VMEM((K, D), jnp.float32),
        pltpu.VMEM((K, D), jnp.float32),
        pltpu.SemaphoreType.DMA,
        pltpu.SemaphoreType.DMA,
        pltpu.SemaphoreType.DMA,
        pltpu.SemaphoreType.DMA,
    ],
)()


ROWS_TC = 1000


def _mlp1_body(x_ref, a0_ref, a1_ref, wa_ref, ba_ref, wb_ref, bb_ref, o_ref):
    h = x_ref[...] + a0_ref[...] + a1_ref[...]
    t = jnp.dot(h, wa_ref[...], preferred_element_type=jnp.float32) + ba_ref[...]
    t = jnp.maximum(t, 0.0)
    u = jnp.dot(t, wb_ref[...], preferred_element_type=jnp.float32) + bb_ref[...]
    o_ref[...] = jnp.maximum(u, 0.0)


def _mlp2_body(x_ref, a0_ref, a1_ref, wa_ref, ba_ref, wb_ref, bb_ref,
               wl_ref, bl_ref, o_ref):
    h = x_ref[...] + a0_ref[...] + a1_ref[...]
    t = jnp.dot(h, wa_ref[...], preferred_element_type=jnp.float32) + ba_ref[...]
    t = jnp.maximum(t, 0.0)
    u = jnp.dot(t, wb_ref[...], preferred_element_type=jnp.float32) + bb_ref[...]
    u = jnp.maximum(u, 0.0)
    o_ref[...] = jnp.dot(u, wl_ref[...], preferred_element_type=jnp.float32) + bl_ref[...]


def _row_spec():
    return pl.BlockSpec((ROWS_TC, D), lambda i: (i, 0))


def _full_spec(shape):
    return pl.BlockSpec(shape, lambda i: (0,) * len(shape))


def _mlp1(x, a0, a1, wa, ba, wb, bb):
    return pl.pallas_call(
        _mlp1_body,
        grid=(N // ROWS_TC,),
        in_specs=[_row_spec(), _row_spec(), _row_spec(),
                  _full_spec((D, D)), _full_spec((1, D)),
                  _full_spec((D, D)), _full_spec((1, D))],
        out_specs=_row_spec(),
        out_shape=jax.ShapeDtypeStruct((N, D), jnp.float32),
    )(x, a0, a1, wa, ba.reshape(1, D), wb, bb.reshape(1, D))


def _mlp2(x, a0, a1, wa, ba, wb, bb, wl_pad, bl_pad):
    return pl.pallas_call(
        _mlp2_body,
        grid=(N // ROWS_TC,),
        in_specs=[_row_spec(), _row_spec(), _row_spec(),
                  _full_spec((D, D)), _full_spec((1, D)),
                  _full_spec((D, D)), _full_spec((1, D)),
                  _full_spec((D, D)), _full_spec((1, D))],
        out_specs=_row_spec(),
        out_shape=jax.ShapeDtypeStruct((N, D), jnp.float32),
    )(x, a0, a1, wa, ba.reshape(1, D), wb, bb.reshape(1, D), wl_pad, bl_pad)


def kernel(x, edge_index, W1a, b1a, W1b, b1b, W2a, b2a, W2b, b2b, Wl, bl):
    order = jnp.argsort(edge_index[0])
    src = edge_index[0][order]
    dst = edge_index[1][order]
    pad = E_PAD - E
    src_p = jnp.concatenate([src, jnp.zeros((pad,), jnp.int32)])
    dump = N + (jnp.arange(pad, dtype=jnp.int32) % (NPAD - N))
    dst_p = jnp.concatenate([dst, dump])
    idx_p = jnp.stack([src_p.reshape(NS, TOT_CH, K),
                       dst_p.reshape(NS, TOT_CH, K)], axis=2)
    zero = jnp.zeros((ROWS_PER_SUB, D), jnp.float32)

    parts1 = _agg_call(x, idx_p, zero)
    h1 = _mlp1(x, parts1[0, :N], parts1[1, :N], W1a, b1a, W1b, b1b)

    parts2 = _agg_call(h1, idx_p, zero)
    wl_pad = jnp.zeros((D, D), jnp.float32).at[:, :OUT].set(Wl)
    bl_pad = jnp.zeros((1, D), jnp.float32).at[0, :OUT].set(bl)
    out_full = _mlp2(h1, parts2[0, :N], parts2[1, :N], W2a, b2a, W2b, b2b,
                     wl_pad, bl_pad)
    return out_full[:, :OUT]


# speedup vs baseline: 1.4202x; 1.4202x over previous
import functools

import jax
import jax.numpy as jnp
from jax import lax
from jax.experimental import pallas as pl
from jax.experimental.pallas import tpu as pltpu
from jax.experimental.pallas import tpu_sc as plsc

N = 10000
D = 128
E = 320000
OUT = 2

NC = 2
NS = 16
NW = NC * NS

K = 64
NBUF = 4
BCH = 20
TOT_CH = 320
CH0 = 160
CH1 = TOT_CH - CH0
E_PAD = NS * TOT_CH * K
NPAD = 10112
ROWS_PER_SUB = NPAD // NS

_sc_mesh = plsc.VectorSubcoreMesh(core_axis_name="c", subcore_axis_name="s")


def _agg_body(table, idx_i, zero_hbm, out, acc, ibuf0, ibuf1,
              rows0, rows1, rows2, rows3,
              isem0, isem1, sem0, sem1, sem2, sem3):
    c = lax.axis_index("c")
    s = lax.axis_index("s")
    pltpu.sync_copy(zero_hbm, acc.at[pl.ds(s * ROWS_PER_SUB, ROWS_PER_SUB)])
    plsc.subcore_barrier()

    ibufs = (ibuf0, ibuf1)
    isems = (isem0, isem1)
    rows = (rows0, rows1, rows2, rows3)
    sems = (sem0, sem1, sem2, sem3)

    def run(ch_lo, ch_hi):
        nblk = (ch_hi - ch_lo) // BCH
        pltpu.async_copy(idx_i.at[s, pl.ds(ch_lo, BCH)], ibuf0, isem0)
        for b in range(nblk):
            ib = ibufs[b % 2]
            pltpu.make_async_copy(idx_i.at[s, pl.ds(ch_lo + b * BCH, BCH)],
                                  ib, isems[b % 2]).wait()
            if b + 1 < nblk:
                pltpu.async_copy(
                    idx_i.at[s, pl.ds(ch_lo + (b + 1) * BCH, BCH)],
                    ibufs[(b + 1) % 2], isems[(b + 1) % 2])

            for q in range(NBUF):
                pltpu.async_copy(table.at[ib.at[q, 0]], rows[q], sems[q])

            def quad(j, carry):
                i0 = NBUF * j
                for q in range(NBUF):
                    pltpu.make_async_copy(table.at[ib.at[i0 + q, 0]],
                                          rows[q], sems[q]).wait()
                    pltpu.sync_copy(rows[q], acc.at[ib.at[i0 + q, 1]], add=True)

                    @pl.when(j < BCH // NBUF - 1)
                    def _():
                        pltpu.async_copy(table.at[ib.at[i0 + NBUF + q, 0]],
                                         rows[q], sems[q])
                return carry

            lax.fori_loop(0, BCH // NBUF, quad, 0)

    if CH0 > 0:
        @pl.when(c == 0)
        def _():
            run(0, CH0)
    if CH1 > 0:
        @pl.when(c == 1)
        def _():
            run(CH0, TOT_CH)

    plsc.subcore_barrier()
    pltpu.sync_copy(acc.at[pl.ds(s * ROWS_PER_SUB, ROWS_PER_SUB)],
                    out.at[c, pl.ds(s * ROWS_PER_SUB, ROWS_PER_SUB)])


_agg_call = functools.partial(
    pl.kernel,
    _agg_body,
    out_type=jax.ShapeDtypeStruct((NC, NPAD, D), jnp.float32),
    mesh=_sc_mesh,
    scratch_types=[
        pltpu.VMEM_SHARED((NPAD, D), jnp.float32),
        pltpu.VMEM((BCH, 2, K), jnp.int32),
        pltpu.VMEM((BCH, 2, K), jnp.int32),
        pltpu.VMEM((K, D), jnp.float32),
        pltpu.VMEM((K, D), jnp.float32),
        pltpu.VMEM((K, D), jnp.float32),
        pltpu.VMEM((K, D), jnp.float32),
        pltpu.SemaphoreType.DMA,
        pltpu.SemaphoreType.DMA,
        pltpu.SemaphoreType.DMA,
        pltpu.SemaphoreType.DMA,
        pltpu.SemaphoreType.DMA,
        pltpu.SemaphoreType.DMA,
    ],
)()


ROWS_TC = 1000


def _mlp1_body(x_ref, a0_ref, a1_ref, wa_ref, ba_ref, wb_ref, bb_ref, o_ref):
    h = x_ref[...] + a0_ref[...] + a1_ref[...]
    t = jnp.dot(h, wa_ref[...], preferred_element_type=jnp.float32) + ba_ref[...]
    t = jnp.maximum(t, 0.0)
    u = jnp.dot(t, wb_ref[...], preferred_element_type=jnp.float32) + bb_ref[...]
    o_ref[...] = jnp.maximum(u, 0.0)


def _mlp2_body(x_ref, a0_ref, a1_ref, wa_ref, ba_ref, wb_ref, bb_ref,
               wl_ref, bl_ref, o_ref):
    h = x_ref[...] + a0_ref[...] + a1_ref[...]
    t = jnp.dot(h, wa_ref[...], preferred_element_type=jnp.float32) + ba_ref[...]
    t = jnp.maximum(t, 0.0)
    u = jnp.dot(t, wb_ref[...], preferred_element_type=jnp.float32) + bb_ref[...]
    u = jnp.maximum(u, 0.0)
    o_ref[...] = jnp.dot(u, wl_ref[...], preferred_element_type=jnp.float32) + bl_ref[...]


def _row_spec():
    return pl.BlockSpec((ROWS_TC, D), lambda i: (i, 0))


def _full_spec(shape):
    return pl.BlockSpec(shape, lambda i: (0,) * len(shape))


def _mlp1(x, a0, a1, wa, ba, wb, bb):
    return pl.pallas_call(
        _mlp1_body,
        grid=(N // ROWS_TC,),
        in_specs=[_row_spec(), _row_spec(), _row_spec(),
                  _full_spec((D, D)), _full_spec((1, D)),
                  _full_spec((D, D)), _full_spec((1, D))],
        out_specs=_row_spec(),
        out_shape=jax.ShapeDtypeStruct((N, D), jnp.float32),
    )(x, a0, a1, wa, ba.reshape(1, D), wb, bb.reshape(1, D))


def _mlp2(x, a0, a1, wa, ba, wb, bb, wl_pad, bl_pad):
    return pl.pallas_call(
        _mlp2_body,
        grid=(N // ROWS_TC,),
        in_specs=[_row_spec(), _row_spec(), _row_spec(),
                  _full_spec((D, D)), _full_spec((1, D)),
                  _full_spec((D, D)), _full_spec((1, D)),
                  _full_spec((D, D)), _full_spec((1, D))],
        out_specs=_row_spec(),
        out_shape=jax.ShapeDtypeStruct((N, D), jnp.float32),
    )(x, a0, a1, wa, ba.reshape(1, D), wb, bb.reshape(1, D), wl_pad, bl_pad)


def kernel(x, edge_index, W1a, b1a, W1b, b1b, W2a, b2a, W2b, b2b, Wl, bl):
    src = edge_index[0]
    dst = edge_index[1]
    pad = E_PAD - E
    src_p = jnp.concatenate([src, jnp.zeros((pad,), jnp.int32)])
    dump = N + (jnp.arange(pad, dtype=jnp.int32) % (NPAD - N))
    dst_p = jnp.concatenate([dst, dump])
    idx_p = jnp.stack([src_p.reshape(NS, TOT_CH, K),
                       dst_p.reshape(NS, TOT_CH, K)], axis=2)
    zero = jnp.zeros((ROWS_PER_SUB, D), jnp.float32)

    parts1 = _agg_call(x, idx_p, zero)
    h1 = _mlp1(x, parts1[0, :N], parts1[1, :N], W1a, b1a, W1b, b1b)

    parts2 = _agg_call(h1, idx_p, zero)
    wl_pad = jnp.zeros((D, D), jnp.float32).at[:, :OUT].set(Wl)
    bl_pad = jnp.zeros((1, D), jnp.float32).at[0, :OUT].set(bl)
    out_full = _mlp2(h1, parts2[0, :N], parts2[1, :N], W2a, b2a, W2b, b2b,
                     wl_pad, bl_pad)
    return out_full[:, :OUT]


# speedup vs baseline: 1.4863x; 1.0466x over previous
import functools

import jax
import jax.numpy as jnp
from jax import lax
from jax.experimental import pallas as pl
from jax.experimental.pallas import tpu as pltpu
from jax.experimental.pallas import tpu_sc as plsc

N = 10000
D = 128
E = 320000
OUT = 2

NC = 2
NS = 16
NW = NC * NS

K = 32
NBUF = 8
BCH = 32
TOT_CH = 640
CH0 = 320
CH1 = TOT_CH - CH0
E_PAD = NS * TOT_CH * K
NPAD = 10112
ROWS_PER_SUB = NPAD // NS

_sc_mesh = plsc.VectorSubcoreMesh(core_axis_name="c", subcore_axis_name="s")


def _agg_body(table, idx_i, zero_hbm, out, acc, ibuf0, ibuf1,
              rows0, rows1, rows2, rows3, rows4, rows5, rows6, rows7,
              isem0, isem1, sem0, sem1, sem2, sem3, sem4, sem5, sem6, sem7):
    c = lax.axis_index("c")
    s = lax.axis_index("s")
    pltpu.sync_copy(zero_hbm, acc.at[pl.ds(s * ROWS_PER_SUB, ROWS_PER_SUB)])
    plsc.subcore_barrier()

    ibufs = (ibuf0, ibuf1)
    isems = (isem0, isem1)
    rows = (rows0, rows1, rows2, rows3, rows4, rows5, rows6, rows7)
    sems = (sem0, sem1, sem2, sem3, sem4, sem5, sem6, sem7)

    def run(ch_lo, ch_hi):
        nblk = (ch_hi - ch_lo) // BCH
        pltpu.async_copy(idx_i.at[s, pl.ds(ch_lo, BCH)], ibuf0, isem0)
        for b in range(nblk):
            ib = ibufs[b % 2]
            pltpu.make_async_copy(idx_i.at[s, pl.ds(ch_lo + b * BCH, BCH)],
                                  ib, isems[b % 2]).wait()
            if b + 1 < nblk:
                pltpu.async_copy(
                    idx_i.at[s, pl.ds(ch_lo + (b + 1) * BCH, BCH)],
                    ibufs[(b + 1) % 2], isems[(b + 1) % 2])

            for q in range(NBUF):
                pltpu.async_copy(table.at[ib.at[q, 0]], rows[q], sems[q])

            def quad(j, carry):
                i0 = NBUF * j
                for q in range(NBUF):
                    pltpu.make_async_copy(table.at[ib.at[i0 + q, 0]],
                                          rows[q], sems[q]).wait()
                    pltpu.sync_copy(rows[q], acc.at[ib.at[i0 + q, 1]], add=True)

                    @pl.when(j < BCH // NBUF - 1)
                    def _():
                        pltpu.async_copy(table.at[ib.at[i0 + NBUF + q, 0]],
                                         rows[q], sems[q])
                return carry

            lax.fori_loop(0, BCH // NBUF, quad, 0)

    if CH0 > 0:
        @pl.when(c == 0)
        def _():
            run(0, CH0)
    if CH1 > 0:
        @pl.when(c == 1)
        def _():
            run(CH0, TOT_CH)

    plsc.subcore_barrier()
    pltpu.sync_copy(acc.at[pl.ds(s * ROWS_PER_SUB, ROWS_PER_SUB)],
                    out.at[c, pl.ds(s * ROWS_PER_SUB, ROWS_PER_SUB)])


_agg_call = functools.partial(
    pl.kernel,
    _agg_body,
    out_type=jax.ShapeDtypeStruct((NC, NPAD, D), jnp.float32),
    mesh=_sc_mesh,
    scratch_types=[
        pltpu.VMEM_SHARED((NPAD, D), jnp.float32),
        pltpu.VMEM((BCH, 2, K), jnp.int32),
        pltpu.VMEM((BCH, 2, K), jnp.int32),
        pltpu.VMEM((K, D), jnp.float32),
        pltpu.VMEM((K, D), jnp.float32),
        pltpu.VMEM((K, D), jnp.float32),
        pltpu.VMEM((K, D), jnp.float32),
        pltpu.VMEM((K, D), jnp.float32),
        pltpu.VMEM((K, D), jnp.float32),
        pltpu.VMEM((K, D), jnp.float32),
        pltpu.VMEM((K, D), jnp.float32),
        pltpu.SemaphoreType.DMA,
        pltpu.SemaphoreType.DMA,
        pltpu.SemaphoreType.DMA,
        pltpu.SemaphoreType.DMA,
        pltpu.SemaphoreType.DMA,
        pltpu.SemaphoreType.DMA,
        pltpu.SemaphoreType.DMA,
        pltpu.SemaphoreType.DMA,
        pltpu.SemaphoreType.DMA,
        pltpu.SemaphoreType.DMA,
    ],
)()


ROWS_TC = 1000


def _mlp1_body(x_ref, a0_ref, a1_ref, wa_ref, ba_ref, wb_ref, bb_ref, o_ref):
    h = x_ref[...] + a0_ref[...] + a1_ref[...]
    t = jnp.dot(h, wa_ref[...], preferred_element_type=jnp.float32) + ba_ref[...]
    t = jnp.maximum(t, 0.0)
    u = jnp.dot(t, wb_ref[...], preferred_element_type=jnp.float32) + bb_ref[...]
    o_ref[...] = jnp.maximum(u, 0.0)


def _mlp2_body(x_ref, a0_ref, a1_ref, wa_ref, ba_ref, wb_ref, bb_ref,
               wl_ref, bl_ref, o_ref):
    h = x_ref[...] + a0_ref[...] + a1_ref[...]
    t = jnp.dot(h, wa_ref[...], preferred_element_type=jnp.float32) + ba_ref[...]
    t = jnp.maximum(t, 0.0)
    u = jnp.dot(t, wb_ref[...], preferred_element_type=jnp.float32) + bb_ref[...]
    u = jnp.maximum(u, 0.0)
    o_ref[...] = jnp.dot(u, wl_ref[...], preferred_element_type=jnp.float32) + bl_ref[...]


def _row_spec():
    return pl.BlockSpec((ROWS_TC, D), lambda i: (i, 0))


def _full_spec(shape):
    return pl.BlockSpec(shape, lambda i: (0,) * len(shape))


def _mlp1(x, a0, a1, wa, ba, wb, bb):
    return pl.pallas_call(
        _mlp1_body,
        grid=(N // ROWS_TC,),
        in_specs=[_row_spec(), _row_spec(), _row_spec(),
                  _full_spec((D, D)), _full_spec((1, D)),
                  _full_spec((D, D)), _full_spec((1, D))],
        out_specs=_row_spec(),
        out_shape=jax.ShapeDtypeStruct((N, D), jnp.float32),
    )(x, a0, a1, wa, ba.reshape(1, D), wb, bb.reshape(1, D))


def _mlp2(x, a0, a1, wa, ba, wb, bb, wl_pad, bl_pad):
    return pl.pallas_call(
        _mlp2_body,
        grid=(N // ROWS_TC,),
        in_specs=[_row_spec(), _row_spec(), _row_spec(),
                  _full_spec((D, D)), _full_spec((1, D)),
                  _full_spec((D, D)), _full_spec((1, D)),
                  _full_spec((D, D)), _full_spec((1, D))],
        out_specs=_row_spec(),
        out_shape=jax.ShapeDtypeStruct((N, D), jnp.float32),
    )(x, a0, a1, wa, ba.reshape(1, D), wb, bb.reshape(1, D), wl_pad, bl_pad)


def kernel(x, edge_index, W1a, b1a, W1b, b1b, W2a, b2a, W2b, b2b, Wl, bl):
    src = edge_index[0]
    dst = edge_index[1]
    pad = E_PAD - E
    src_p = jnp.concatenate([src, jnp.zeros((pad,), jnp.int32)])
    dump = N + (jnp.arange(pad, dtype=jnp.int32) % (NPAD - N))
    dst_p = jnp.concatenate([dst, dump])
    idx_p = jnp.stack([src_p.reshape(NS, TOT_CH, K),
                       dst_p.reshape(NS, TOT_CH, K)], axis=2)
    zero = jnp.zeros((ROWS_PER_SUB, D), jnp.float32)

    parts1 = _agg_call(x, idx_p, zero)
    h1 = _mlp1(x, parts1[0, :N], parts1[1, :N], W1a, b1a, W1b, b1b)

    parts2 = _agg_call(h1, idx_p, zero)
    wl_pad = jnp.zeros((D, D), jnp.float32).at[:, :OUT].set(Wl)
    bl_pad = jnp.zeros((1, D), jnp.float32).at[0, :OUT].set(bl)
    out_full = _mlp2(h1, parts2[0, :N], parts2[1, :N], W2a, b2a, W2b, b2b,
                     wl_pad, bl_pad)
    return out_full[:, :OUT]


# speedup vs baseline: 4.4128x; 2.9690x over previous
import functools

import jax
import jax.numpy as jnp
from jax import lax
from jax.experimental import pallas as pl
from jax.experimental.pallas import tpu as pltpu
from jax.experimental.pallas import tpu_sc as plsc

N = 10000
D = 128
E = 320000
OUT = 2

NC = 2
NS = 16
NW = NC * NS

K = 32
NBUF = 8
BCH = 32
TOT_CH = 640
CH0 = 320
CH1 = TOT_CH - CH0
E_PAD = NS * TOT_CH * K
NPAD = 10112
ROWS_PER_SUB = NPAD // NS

_sc_mesh = plsc.VectorSubcoreMesh(core_axis_name="c", subcore_axis_name="s")


def _agg_body(table, idx_i, zero_hbm, out, acc, ibuf0, ibuf1,
              rows0, rows1, rows2, rows3, rows4, rows5, rows6, rows7,
              isem0, isem1, sem0, sem1, sem2, sem3, sem4, sem5, sem6, sem7):
    c = lax.axis_index("c")
    s = lax.axis_index("s")
    pltpu.sync_copy(zero_hbm, acc.at[pl.ds(s * ROWS_PER_SUB, ROWS_PER_SUB)])
    plsc.subcore_barrier()

    ibufs = (ibuf0, ibuf1)
    isems = (isem0, isem1)
    rows = (rows0, rows1, rows2, rows3, rows4, rows5, rows6, rows7)
    sems = (sem0, sem1, sem2, sem3, sem4, sem5, sem6, sem7)

    def run(ch_lo, ch_hi):
        nblk = (ch_hi - ch_lo) // BCH
        pltpu.async_copy(idx_i.at[s, pl.ds(ch_lo, BCH)], ibuf0, isem0)
        for b in range(nblk):
            ib = ibufs[b % 2]
            pltpu.make_async_copy(idx_i.at[s, pl.ds(ch_lo + b * BCH, BCH)],
                                  ib, isems[b % 2]).wait()
            if b + 1 < nblk:
                pltpu.async_copy(
                    idx_i.at[s, pl.ds(ch_lo + (b + 1) * BCH, BCH)],
                    ibufs[(b + 1) % 2], isems[(b + 1) % 2])

            for q in range(NBUF):
                pltpu.async_copy(table.at[ib.at[q, 0]], rows[q], sems[q])

            def quad(j, carry):
                i0 = NBUF * j
                for q in range(NBUF):
                    pltpu.make_async_copy(table.at[ib.at[i0 + q, 0]],
                                          rows[q], sems[q]).wait()
                    pltpu.sync_copy(rows[q], acc.at[ib.at[i0 + q, 1]], add=True)

                    @pl.when(j < BCH // NBUF - 1)
                    def _():
                        pltpu.async_copy(table.at[ib.at[i0 + NBUF + q, 0]],
                                         rows[q], sems[q])
                return carry

            lax.fori_loop(0, BCH // NBUF, quad, 0)

    if CH0 > 0:
        @pl.when(c == 0)
        def _():
            run(0, CH0)
    if CH1 > 0:
        @pl.when(c == 1)
        def _():
            run(CH0, TOT_CH)

    plsc.subcore_barrier()
    pltpu.sync_copy(acc.at[pl.ds(s * ROWS_PER_SUB, ROWS_PER_SUB)],
                    out.at[c, pl.ds(s * ROWS_PER_SUB, ROWS_PER_SUB)])


_agg_call = functools.partial(
    pl.kernel,
    _agg_body,
    out_type=jax.ShapeDtypeStruct((NC, NPAD, D), jnp.float32),
    mesh=_sc_mesh,
    scratch_types=[
        pltpu.VMEM_SHARED((NPAD, D), jnp.float32),
        pltpu.VMEM((BCH, 2, K), jnp.int32),
        pltpu.VMEM((BCH, 2, K), jnp.int32),
        pltpu.VMEM((K, D), jnp.float32),
        pltpu.VMEM((K, D), jnp.float32),
        pltpu.VMEM((K, D), jnp.float32),
        pltpu.VMEM((K, D), jnp.float32),
        pltpu.VMEM((K, D), jnp.float32),
        pltpu.VMEM((K, D), jnp.float32),
        pltpu.VMEM((K, D), jnp.float32),
        pltpu.VMEM((K, D), jnp.float32),
        pltpu.SemaphoreType.DMA,
        pltpu.SemaphoreType.DMA,
        pltpu.SemaphoreType.DMA,
        pltpu.SemaphoreType.DMA,
        pltpu.SemaphoreType.DMA,
        pltpu.SemaphoreType.DMA,
        pltpu.SemaphoreType.DMA,
        pltpu.SemaphoreType.DMA,
        pltpu.SemaphoreType.DMA,
        pltpu.SemaphoreType.DMA,
    ],
)()


ROWS_TC = 1000


def _mlp1_body(x_ref, a0_ref, a1_ref, wa_ref, ba_ref, wb_ref, bb_ref, o_ref):
    h = x_ref[...] + a0_ref[...] + a1_ref[...]
    t = jnp.dot(h, wa_ref[...], preferred_element_type=jnp.float32) + ba_ref[...]
    t = jnp.maximum(t, 0.0)
    u = jnp.dot(t, wb_ref[...], preferred_element_type=jnp.float32) + bb_ref[...]
    o_ref[...] = jnp.maximum(u, 0.0)


def _mlp2_body(x_ref, a0_ref, a1_ref, wa_ref, ba_ref, wb_ref, bb_ref,
               wl_ref, bl_ref, o_ref):
    h = x_ref[...] + a0_ref[...] + a1_ref[...]
    t = jnp.dot(h, wa_ref[...], preferred_element_type=jnp.float32) + ba_ref[...]
    t = jnp.maximum(t, 0.0)
    u = jnp.dot(t, wb_ref[...], preferred_element_type=jnp.float32) + bb_ref[...]
    u = jnp.maximum(u, 0.0)
    o_ref[...] = jnp.dot(u, wl_ref[...], preferred_element_type=jnp.float32) + bl_ref[...]


def _row_spec():
    return pl.BlockSpec((ROWS_TC, D), lambda i: (i, 0))


def _full_spec(shape):
    return pl.BlockSpec(shape, lambda i: (0,) * len(shape))


def _mlp1(x, a0, a1, wa, ba, wb, bb):
    return pl.pallas_call(
        _mlp1_body,
        grid=(N // ROWS_TC,),
        in_specs=[_row_spec(), _row_spec(), _row_spec(),
                  _full_spec((D, D)), _full_spec((1, D)),
                  _full_spec((D, D)), _full_spec((1, D))],
        out_specs=_row_spec(),
        out_shape=jax.ShapeDtypeStruct((N, D), jnp.float32),
    )(x, a0, a1, wa, ba.reshape(1, D), wb, bb.reshape(1, D))


def _mlp2(x, a0, a1, wa, ba, wb, bb, wl_pad, bl_pad):
    return pl.pallas_call(
        _mlp2_body,
        grid=(N // ROWS_TC,),
        in_specs=[_row_spec(), _row_spec(), _row_spec(),
                  _full_spec((D, D)), _full_spec((1, D)),
                  _full_spec((D, D)), _full_spec((1, D)),
                  _full_spec((D, D)), _full_spec((1, D))],
        out_specs=_row_spec(),
        out_shape=jax.ShapeDtypeStruct((N, D), jnp.float32),
    )(x, a0, a1, wa, ba.reshape(1, D), wb, bb.reshape(1, D), wl_pad, bl_pad)


def kernel(x, edge_index, W1a, b1a, W1b, b1b, W2a, b2a, W2b, b2b, Wl, bl):
    src = edge_index[0]
    dst = edge_index[1]
    pad = E_PAD - E
    pad_src = jnp.arange(pad, dtype=jnp.int32) % N
    src_p = jnp.concatenate([src, pad_src])
    dump = N + (jnp.arange(pad, dtype=jnp.int32) % (NPAD - N))
    dst_p = jnp.concatenate([dst, dump])
    idx_p = jnp.stack([src_p.reshape(NS, TOT_CH, K),
                       dst_p.reshape(NS, TOT_CH, K)], axis=2)
    zero = jnp.zeros((ROWS_PER_SUB, D), jnp.float32)

    parts1 = _agg_call(x, idx_p, zero)
    h1 = _mlp1(x, parts1[0, :N], parts1[1, :N], W1a, b1a, W1b, b1b)

    parts2 = _agg_call(h1, idx_p, zero)
    wl_pad = jnp.zeros((D, D), jnp.float32).at[:, :OUT].set(Wl)
    bl_pad = jnp.zeros((1, D), jnp.float32).at[0, :OUT].set(bl)
    out_full = _mlp2(h1, parts2[0, :N], parts2[1, :N], W2a, b2a, W2b, b2b,
                     wl_pad, bl_pad)
    return out_full[:, :OUT]


# speedup vs baseline: 4.5961x; 1.0415x over previous
import functools

import jax
import jax.numpy as jnp
from jax import lax
from jax.experimental import pallas as pl
from jax.experimental.pallas import tpu as pltpu
from jax.experimental.pallas import tpu_sc as plsc

N = 10000
D = 128
E = 320000
OUT = 2

NC = 2
NS = 16
NW = NC * NS

K = 32
NBUF = 8
BCH = 32
TOT_CH = 640
CH0 = 320
CH1 = TOT_CH - CH0
E_PAD = NS * TOT_CH * K
NPAD = 10112
ROWS_PER_SUB = NPAD // NS

_sc_mesh = plsc.VectorSubcoreMesh(core_axis_name="c", subcore_axis_name="s")


def _agg_body(table, idx_i, zero_hbm, out, acc, ibuf0, ibuf1,
              rows0, rows1, rows2, rows3, rows4, rows5, rows6, rows7,
              isem0, isem1, sem0, sem1, sem2, sem3, sem4, sem5, sem6, sem7):
    c = lax.axis_index("c")
    s = lax.axis_index("s")
    pltpu.sync_copy(zero_hbm, acc.at[pl.ds(s * ROWS_PER_SUB, ROWS_PER_SUB)])
    plsc.subcore_barrier()

    ibufs = (ibuf0, ibuf1)
    isems = (isem0, isem1)
    rows = (rows0, rows1, rows2, rows3, rows4, rows5, rows6, rows7)
    sems = (sem0, sem1, sem2, sem3, sem4, sem5, sem6, sem7)

    def run(ch_lo, ch_hi):
        nblk = (ch_hi - ch_lo) // BCH
        pltpu.async_copy(idx_i.at[s, pl.ds(ch_lo, BCH)], ibuf0, isem0)
        for b in range(nblk):
            ib = ibufs[b % 2]
            pltpu.make_async_copy(idx_i.at[s, pl.ds(ch_lo + b * BCH, BCH)],
                                  ib, isems[b % 2]).wait()
            if b + 1 < nblk:
                pltpu.async_copy(
                    idx_i.at[s, pl.ds(ch_lo + (b + 1) * BCH, BCH)],
                    ibufs[(b + 1) % 2], isems[(b + 1) % 2])

            for q in range(NBUF):
                pltpu.async_copy(table.at[ib.at[q, 0]], rows[q], sems[q])

            def quad(j, carry):
                i0 = NBUF * j
                for q in range(NBUF):
                    pltpu.make_async_copy(table.at[ib.at[i0 + q, 0]],
                                          rows[q], sems[q]).wait()
                    pltpu.sync_copy(rows[q], acc.at[ib.at[i0 + q, 1]], add=True)

                    @pl.when(j < BCH // NBUF - 1)
                    def _():
                        pltpu.async_copy(table.at[ib.at[i0 + NBUF + q, 0]],
                                         rows[q], sems[q])
                return carry

            lax.fori_loop(0, BCH // NBUF, quad, 0)

    if CH0 > 0:
        @pl.when(c == 0)
        def _():
            run(0, CH0)
    if CH1 > 0:
        @pl.when(c == 1)
        def _():
            run(CH0, TOT_CH)

    plsc.subcore_barrier()
    pltpu.sync_copy(acc.at[pl.ds(s * ROWS_PER_SUB, ROWS_PER_SUB)],
                    out.at[c, pl.ds(s * ROWS_PER_SUB, ROWS_PER_SUB)])


_agg_call = functools.partial(
    pl.kernel,
    _agg_body,
    out_type=jax.ShapeDtypeStruct((NC, NPAD, D), jnp.float32),
    mesh=_sc_mesh,
    scratch_types=[
        pltpu.VMEM_SHARED((NPAD, D), jnp.float32),
        pltpu.VMEM((BCH, 2, K), jnp.int32),
        pltpu.VMEM((BCH, 2, K), jnp.int32),
        pltpu.VMEM((K, D), jnp.float32),
        pltpu.VMEM((K, D), jnp.float32),
        pltpu.VMEM((K, D), jnp.float32),
        pltpu.VMEM((K, D), jnp.float32),
        pltpu.VMEM((K, D), jnp.float32),
        pltpu.VMEM((K, D), jnp.float32),
        pltpu.VMEM((K, D), jnp.float32),
        pltpu.VMEM((K, D), jnp.float32),
        pltpu.SemaphoreType.DMA,
        pltpu.SemaphoreType.DMA,
        pltpu.SemaphoreType.DMA,
        pltpu.SemaphoreType.DMA,
        pltpu.SemaphoreType.DMA,
        pltpu.SemaphoreType.DMA,
        pltpu.SemaphoreType.DMA,
        pltpu.SemaphoreType.DMA,
        pltpu.SemaphoreType.DMA,
        pltpu.SemaphoreType.DMA,
    ],
)()


ROWS_TC = 1000


def _mlp1_body(x_ref, a0_ref, a1_ref, wa_ref, ba_ref, wb_ref, bb_ref, o_ref):
    h = x_ref[...] + a0_ref[0] + a1_ref[0]
    t = jnp.dot(h, wa_ref[...], preferred_element_type=jnp.float32) + ba_ref[...]
    t = jnp.maximum(t, 0.0)
    u = jnp.dot(t, wb_ref[...], preferred_element_type=jnp.float32) + bb_ref[...]
    o_ref[...] = jnp.maximum(u, 0.0)


def _mlp2_body(x_ref, a0_ref, a1_ref, wa_ref, ba_ref, wb_ref, bb_ref,
               wl_ref, bl_ref, o_ref):
    h = x_ref[...] + a0_ref[0] + a1_ref[0]
    t = jnp.dot(h, wa_ref[...], preferred_element_type=jnp.float32) + ba_ref[...]
    t = jnp.maximum(t, 0.0)
    u = jnp.dot(t, wb_ref[...], preferred_element_type=jnp.float32) + bb_ref[...]
    u = jnp.maximum(u, 0.0)
    o_ref[...] = jnp.dot(u, wl_ref[...], preferred_element_type=jnp.float32) + bl_ref[...]


def _row_spec():
    return pl.BlockSpec((ROWS_TC, D), lambda i: (i, 0))


def _part_spec(p):
    return pl.BlockSpec((1, ROWS_TC, D), lambda i, p=p: (p, i, 0))


def _full_spec(shape):
    return pl.BlockSpec(shape, lambda i: (0,) * len(shape))


def _mlp1(x, parts, wa, ba, wb, bb):
    return pl.pallas_call(
        _mlp1_body,
        grid=(N // ROWS_TC,),
        in_specs=[_row_spec(), _part_spec(0), _part_spec(1),
                  _full_spec((D, D)), _full_spec((1, D)),
                  _full_spec((D, D)), _full_spec((1, D))],
        out_specs=_row_spec(),
        out_shape=jax.ShapeDtypeStruct((N, D), jnp.float32),
    )(x, parts, parts, wa, ba.reshape(1, D), wb, bb.reshape(1, D))


def _mlp2(x, parts, wa, ba, wb, bb, wl_pad, bl_pad):
    return pl.pallas_call(
        _mlp2_body,
        grid=(N // ROWS_TC,),
        in_specs=[_row_spec(), _part_spec(0), _part_spec(1),
                  _full_spec((D, D)), _full_spec((1, D)),
                  _full_spec((D, D)), _full_spec((1, D)),
                  _full_spec((D, D)), _full_spec((1, D))],
        out_specs=_row_spec(),
        out_shape=jax.ShapeDtypeStruct((N, D), jnp.float32),
    )(x, parts, parts, wa, ba.reshape(1, D), wb, bb.reshape(1, D), wl_pad, bl_pad)


def kernel(x, edge_index, W1a, b1a, W1b, b1b, W2a, b2a, W2b, b2b, Wl, bl):
    src = edge_index[0]
    dst = edge_index[1]
    pad = E_PAD - E
    pad_src = jnp.arange(pad, dtype=jnp.int32) % N
    src_p = jnp.concatenate([src, pad_src])
    dump = N + (jnp.arange(pad, dtype=jnp.int32) % (NPAD - N))
    dst_p = jnp.concatenate([dst, dump])
    idx_p = jnp.stack([src_p, dst_p]).reshape(2, NS, TOT_CH, K).transpose(1, 2, 0, 3)
    zero = jnp.zeros((ROWS_PER_SUB, D), jnp.float32)

    parts1 = _agg_call(x, idx_p, zero)
    h1 = _mlp1(x, parts1, W1a, b1a, W1b, b1b)

    parts2 = _agg_call(h1, idx_p, zero)
    wl_pad = jnp.zeros((D, D), jnp.float32).at[:, :OUT].set(Wl)
    bl_pad = jnp.zeros((1, D), jnp.float32).at[0, :OUT].set(bl)
    out_full = _mlp2(h1, parts2, W2a, b2a, W2b, b2b, wl_pad, bl_pad)
    return out_full[:, :OUT]
